# baseline (speedup 1.0000x reference)
import functools

import jax
import jax.numpy as jnp
from jax import lax
from jax.experimental import pallas as pl
from jax.experimental.pallas import tpu as pltpu
from jax.experimental.pallas import tpu_sc as plsc

_B, _C = 2, 96
_OH = _OW = 27
_KH = _KW = 16
_H = _W = 224
_N_IMG = _B * _C
_N_WORKERS = 32
_IMGS_PER_WORKER = _N_IMG // _N_WORKERS


def _fold_sc(x):
    mesh = plsc.VectorSubcoreMesh(core_axis_name="c", subcore_axis_name="s")

    @functools.partial(
        pl.kernel,
        out_type=jax.ShapeDtypeStruct((_B, _C, _H, _W), jnp.float32),
        mesh=mesh,
        scratch_types=[
            pltpu.VMEM((_OW, _KH, _KW), jnp.float32),
            pltpu.VMEM((_H, _W), jnp.float32),
        ],
    )
    def k(x_hbm, out_hbm, xrow, obuf):
        wid = lax.axis_index("s") * 2 + lax.axis_index("c")
        zeros16 = jnp.zeros((16,), jnp.float32)

        def zero_body(r, carry):
            for u in range(_W // 16):
                obuf[r, pl.ds(u * 16, 16)] = zeros16
            return carry

        def run_image(img):
            b = img // _C
            c = img % _C
            lax.fori_loop(0, _H, zero_body, 0)

            def row_body(i, carry):
                pltpu.sync_copy(x_hbm.at[b, c, i], xrow)

                def kh_body(kh, inner):
                    row = 8 * i + kh
                    for j in range(_OW):
                        v = xrow[j, kh]
                        plsc.addupdate(obuf.at[row, pl.ds(8 * j, 16)], v)
                    return inner

                lax.fori_loop(0, _KH, kh_body, 0)
                return carry

            lax.fori_loop(0, _OH, row_body, 0)
            pltpu.sync_copy(obuf, out_hbm.at[b, c])

        for m in range(_IMGS_PER_WORKER):
            run_image(wid * _IMGS_PER_WORKER + m)

    return k(x)


def kernel(x):
    return _fold_sc(x)


# speedup vs baseline: 1.3914x; 1.3914x over previous
import functools

import jax
import jax.numpy as jnp
from jax import lax
from jax.experimental import pallas as pl
from jax.experimental.pallas import tpu as pltpu
from jax.experimental.pallas import tpu_sc as plsc

_B, _C = 2, 96
_OH = _OW = 27
_KH = _KW = 16
_H = _W = 224
_N_IMG = _B * _C
_N_WORKERS = 32
_IMGS_PER_WORKER = _N_IMG // _N_WORKERS


_ROW_ELEMS = _OW * _KH * _KW


def _fold_sc(xr):
    mesh = plsc.VectorSubcoreMesh(core_axis_name="c", subcore_axis_name="s")

    @functools.partial(
        pl.kernel,
        out_type=jax.ShapeDtypeStruct((_B, _C, _H, _W), jnp.float32),
        mesh=mesh,
        scratch_types=[
            pltpu.VMEM((_ROW_ELEMS,), jnp.float32),
            pltpu.VMEM((_H, _W), jnp.float32),
        ],
    )
    def k(x_hbm, out_hbm, xrow, obuf):
        wid = lax.axis_index("s") * 2 + lax.axis_index("c")
        zeros16 = jnp.zeros((16,), jnp.float32)

        def zero_body(r, carry):
            for u in range(_W // 16):
                obuf[r, pl.ds(u * 16, 16)] = zeros16
            return carry

        def run_image(img):
            b = img // _C
            c = img % _C
            lax.fori_loop(0, _H, zero_body, 0)

            def row_body(i, carry):
                pltpu.sync_copy(x_hbm.at[img, i], xrow)

                def kh_body(kh, inner):
                    row = 8 * i + kh
                    base_src = kh * _KW
                    for j in range(_OW):
                        v = xrow[pl.ds(base_src + j * (_KH * _KW), 16)]
                        plsc.addupdate(obuf.at[row, pl.ds(8 * j, 16)], v)
                    return inner

                lax.fori_loop(0, _KH, kh_body, 0)
                return carry

            lax.fori_loop(0, _OH, row_body, 0)
            pltpu.sync_copy(obuf, out_hbm.at[b, c])

        for m in range(_IMGS_PER_WORKER):
            run_image(wid * _IMGS_PER_WORKER + m)

    return k(xr)


def kernel(x):
    xr = x.reshape(_N_IMG, _OH, _ROW_ELEMS)
    return _fold_sc(xr)
